# baseline (speedup 1.0000x reference)
import functools

import jax
import jax.numpy as jnp
from jax import lax
from jax.experimental import pallas as pl
from jax.experimental.pallas import tpu as pltpu
from jax.experimental.pallas import tpu_sc as plsc

F = 26
V = 100000
D = 64
B = 4096
NC = 2
NS = 16
NW = NC * NS
ROWS = B * F
RPW = ROWS // NW
CHUNK = 128
NCH = RPW // CHUNK

_mesh = plsc.VectorSubcoreMesh(core_axis_name="c", subcore_axis_name="s")


@functools.partial(
    pl.kernel,
    mesh=_mesh,
    compiler_params=pltpu.CompilerParams(needs_layout_passes=False),
    out_type=jax.ShapeDtypeStruct((ROWS // 2, 2 * D), jnp.float32),
    scratch_types=[
        pltpu.VMEM((RPW,), jnp.int32),
        pltpu.VMEM((RPW,), jnp.int32),
        pltpu.VMEM((NCH, CHUNK), jnp.int32),
        pltpu.VMEM((CHUNK, 2 * D), jnp.float32),
        pltpu.VMEM((CHUNK // 2, 2 * D), jnp.float32),
        pltpu.SemaphoreType.DMA,
    ],
)
def _emb_gather(x_hbm, off_hbm, tab_hbm, out_hbm,
                xv, offv, idxv, rows, outb, sem):
    wid = lax.axis_index("s") * NC + lax.axis_index("c")
    base = pl.multiple_of(wid * RPW, RPW)
    pltpu.sync_copy(x_hbm.at[pl.ds(base, RPW)], xv)
    pltpu.sync_copy(off_hbm, offv)
    for j in range(NCH):
        for k in range(CHUNK // 16):
            s = j * CHUNK + k * 16
            e = xv[pl.ds(s, 16)] + offv[pl.ds(s, 16)]
            idxv[j, pl.ds(k * 16, 16)] = lax.shift_right_logical(e, 1)
    lane = lax.iota(jnp.int32, 16)
    for j in range(NCH):
        pltpu.async_copy(tab_hbm.at[idxv.at[j]], rows, sem).wait()

        def body(g, _, j=j):
            ii = g * 16 + lane
            h = (xv[pl.ds(j * CHUNK + g * 16, 16)] & 1) * D
            orow = lax.shift_right_logical(ii, 1)
            ocol = (ii & 1) * D
            for c in range(D):
                v = plsc.load_gather(rows, [ii, h + c])
                plsc.store_scatter(outb, [orow, ocol + c], v)
            return _

        lax.fori_loop(0, CHUNK // 16, body, None)
        obase = pl.multiple_of((base + j * CHUNK) // 2, CHUNK // 2)
        pltpu.sync_copy(outb, out_hbm.at[pl.ds(obase, CHUNK // 2)])


def kernel(x, tables):
    xf = x.reshape(ROWS).astype(jnp.int32)
    tf = tables.reshape(F * V // 2, 2 * D)
    off = jnp.tile(jnp.arange(F, dtype=jnp.int32) * V, RPW // F)
    out = _emb_gather(xf, off, tf)
    return out.reshape(B, F * D)


# speedup vs baseline: 1.3744x; 1.3744x over previous
import functools

import jax
import jax.numpy as jnp
from jax import lax
from jax.experimental import pallas as pl
from jax.experimental.pallas import tpu as pltpu
from jax.experimental.pallas import tpu_sc as plsc

F = 26
V = 100000
D = 64
B = 4096
NC = 2
NS = 16
NW = NC * NS
ROWS = B * F
RPW = ROWS // NW
CHUNK = 128
NCH = RPW // CHUNK
NBUF = 16

_mesh = plsc.VectorSubcoreMesh(core_axis_name="c", subcore_axis_name="s")


@functools.partial(
    pl.kernel,
    mesh=_mesh,
    compiler_params=pltpu.CompilerParams(needs_layout_passes=False),
    out_type=jax.ShapeDtypeStruct((ROWS // 2, 2 * D), jnp.float32),
    scratch_types=[
        pltpu.VMEM((RPW,), jnp.int32),
        pltpu.VMEM((RPW,), jnp.int32),
        pltpu.VMEM((RPW,), jnp.int32),
        pltpu.VMEM((RPW,), jnp.int32),
        pltpu.VMEM((NBUF, 8, D), jnp.float32),
        pltpu.VMEM((CHUNK // 2, 2 * D), jnp.float32),
        pltpu.SemaphoreType.DMA((NBUF,)),
    ],
)
def _emb_gather(x_hbm, f_hbm, tab_hbm, out_hbm,
                xv, fv, tv, rv, blk, outb, sems):
    wid = lax.axis_index("s") * NC + lax.axis_index("c")
    base = pl.multiple_of(wid * RPW, RPW)
    pltpu.sync_copy(x_hbm.at[pl.ds(base, RPW)], xv)
    pltpu.sync_copy(f_hbm, fv)
    for k in range(RPW // 16):
        s = k * 16
        xk = xv[pl.ds(s, 16)]
        tv[pl.ds(s, 16)] = lax.shift_right_logical(xk, 3)
        rv[pl.ds(s, 16)] = xk & 7
    lanes = lax.iota(jnp.int32, 16)

    def chunk_body(j, _):
        c0 = j * CHUNK

        def scal(vec, i):
            v = vec[pl.ds(c0 + (i // 16) * 16, 16)]
            return jnp.sum(jnp.where(lanes == (i % 16), v, 0))

        def issue(i):
            t8 = pl.multiple_of(scal(tv, i) * 8, 8)
            return pltpu.async_copy(
                tab_hbm.at[scal(fv, i), pl.ds(t8, 8), :],
                blk.at[i % NBUF], sems.at[i % NBUF])

        for i in range(NBUF):
            issue(i)
        for i in range(CHUNK):
            slot = i % NBUF
            pltpu.make_async_copy(
                tab_hbm.at[0, pl.ds(0, 8), :], blk.at[slot],
                sems.at[slot]).wait()
            r = scal(rv, i)
            orow = i >> 1
            ocol = (i & 1) * D
            for k in range(D // 16):
                outb[orow, pl.ds(ocol + k * 16, 16)] = blk[slot, r,
                                                           pl.ds(k * 16, 16)]
            if i + NBUF < CHUNK:
                issue(i + NBUF)
        obase = pl.multiple_of((base + c0) // 2, CHUNK // 2)
        pltpu.sync_copy(outb, out_hbm.at[pl.ds(obase, CHUNK // 2)])
        return _

    lax.fori_loop(0, NCH, chunk_body, None)


def kernel(x, tables):
    xf = x.reshape(ROWS).astype(jnp.int32)
    fvec = jnp.tile(jnp.arange(F, dtype=jnp.int32), RPW // F)
    out = _emb_gather(xf, fvec, tables)
    return out.reshape(B, F * D)


# speedup vs baseline: 1.9871x; 1.4458x over previous
import functools

import jax
import jax.numpy as jnp
from jax import lax
from jax.experimental import pallas as pl
from jax.experimental.pallas import tpu as pltpu
from jax.experimental.pallas import tpu_sc as plsc

F = 26
V = 100000
D = 64
B = 4096
NC = 2
NS = 16
NW = NC * NS
ROWS = B * F
RPW = ROWS // NW
CHUNK = 128
NCH = RPW // CHUNK
NBUF = 16

_mesh = plsc.VectorSubcoreMesh(core_axis_name="c", subcore_axis_name="s")


@functools.partial(
    pl.kernel,
    mesh=_mesh,
    compiler_params=pltpu.CompilerParams(needs_layout_passes=False),
    out_type=jax.ShapeDtypeStruct((ROWS // 2, 2 * D), jnp.float32),
    scratch_types=[
        pltpu.VMEM((RPW,), jnp.int32),
        pltpu.VMEM((RPW,), jnp.int32),
        pltpu.VMEM((RPW,), jnp.int32),
        pltpu.VMEM((RPW,), jnp.int32),
        pltpu.VMEM((NBUF, 8, D), jnp.float32),
        pltpu.VMEM((CHUNK // 2, 2 * D), jnp.float32),
        pltpu.SemaphoreType.DMA((NBUF,)),
    ],
)
def _emb_gather(x_hbm, off_hbm, tab_hbm, out_hbm,
                xv, offv, tv, rv, blk, outb, sems):
    wid = lax.axis_index("s") * NC + lax.axis_index("c")
    base = pl.multiple_of(wid * RPW, RPW)
    pltpu.sync_copy(x_hbm.at[pl.ds(base, RPW)], xv)
    pltpu.sync_copy(off_hbm, offv)
    for k in range(RPW // 16):
        s = k * 16
        ek = xv[pl.ds(s, 16)] + offv[pl.ds(s, 16)]
        tv[pl.ds(s, 16)] = lax.shift_right_logical(ek, 3)
        rv[pl.ds(s, 16)] = ek & 7
    lanes = lax.iota(jnp.int32, 16)

    def chunk_body(j, _):
        c0 = j * CHUNK

        def scal(vec, i):
            v = vec[pl.ds(c0 + (i // 16) * 16, 16)]
            return jnp.sum(jnp.where(lanes == (i % 16), v, 0))

        def issue(i):
            g8 = pl.multiple_of(scal(tv, i) * 8, 8)
            return pltpu.async_copy(
                tab_hbm.at[pl.ds(g8, 8), :],
                blk.at[i % NBUF], sems.at[i % NBUF])

        for i in range(NBUF):
            issue(i)
        for i in range(CHUNK):
            slot = i % NBUF
            pltpu.make_async_copy(
                tab_hbm.at[pl.ds(0, 8), :], blk.at[slot],
                sems.at[slot]).wait()
            r = scal(rv, i)
            orow = i >> 1
            ocol = (i & 1) * D
            for k in range(D // 16):
                outb[orow, pl.ds(ocol + k * 16, 16)] = blk[slot, r,
                                                           pl.ds(k * 16, 16)]
            if i + NBUF < CHUNK:
                issue(i + NBUF)
        obase = pl.multiple_of((base + c0) // 2, CHUNK // 2)
        pltpu.sync_copy(outb, out_hbm.at[pl.ds(obase, CHUNK // 2)])
        return _

    lax.fori_loop(0, NCH, chunk_body, None)


def kernel(x, tables):
    xf = x.reshape(ROWS).astype(jnp.int32)
    tf = tables.reshape(F * V, D)
    off = jnp.tile(jnp.arange(F, dtype=jnp.int32) * V, RPW // F)
    out = _emb_gather(xf, off, tf)
    return out.reshape(B, F * D)
